# Initial kernel scaffold; baseline (speedup 1.0000x reference)
import jax
import jax.numpy as jnp
from jax.experimental import pallas as pl


def kernel(ids, table, A, Bm):
    raise NotImplementedError("write your pallas kernel here")


# speedup vs baseline: 4.6679x; 4.6679x over previous
import functools

import jax
import jax.numpy as jnp
from jax import lax
from jax.experimental import pallas as pl
from jax.experimental.pallas import tpu as pltpu
from jax.experimental.pallas import tpu_sc as plsc

SCALE = 0.5
CHUNK = 128


def _fuse_table(table, A, Bm):
    V, H = table.shape
    R = A.shape[1]
    blk = 2000
    assert V % blk == 0

    def body(t_ref, a_ref, b_ref, o_ref):
        delta = jax.lax.dot_general(
            a_ref[...], b_ref[...], (((1,), (0,)), ((), ())),
            preferred_element_type=jnp.float32)
        o_ref[...] = t_ref[...] + delta * SCALE

    return pl.pallas_call(
        body,
        grid=(V // blk,),
        in_specs=[
            pl.BlockSpec((blk, H), lambda i: (i, 0)),
            pl.BlockSpec((blk, R), lambda i: (i, 0)),
            pl.BlockSpec((R, H), lambda i: (0, 0)),
        ],
        out_specs=pl.BlockSpec((blk, H), lambda i: (i, 0)),
        out_shape=jax.ShapeDtypeStruct((V, H), jnp.float32),
    )(table, A, Bm)


@functools.lru_cache(maxsize=None)
def _make_gather(V, H, BT):
    info = plsc.get_sparse_core_info()
    NC, NS = info.num_cores, info.num_subcores
    NW = NC * NS
    assert BT % (NW * CHUNK) == 0
    per_tile = BT // NW
    n_chunks = per_tile // CHUNK
    mesh = plsc.VectorSubcoreMesh(core_axis_name="c", subcore_axis_name="s")

    @functools.partial(
        pl.kernel,
        mesh=mesh,
        out_type=jax.ShapeDtypeStruct((BT, H), jnp.float32),
        scratch_types=[
            pltpu.VMEM((n_chunks, CHUNK), jnp.int32),
            pltpu.VMEM((CHUNK, H), jnp.float32),
            pltpu.SemaphoreType.DMA,
        ],
    )
    def gather_k(fused_hbm, ids_hbm, out_hbm, idx_v, rows_v, sem):
        c = lax.axis_index("c")
        s = lax.axis_index("s")
        wid = s * NC + c
        base = wid * per_tile
        pltpu.sync_copy(ids_hbm.at[wid], idx_v)

        def chunk_body(j, carry):
            pltpu.async_copy(fused_hbm.at[idx_v.at[j]], rows_v, sem).wait()
            pltpu.sync_copy(rows_v, out_hbm.at[pl.ds(base + j * CHUNK, CHUNK)])
            return carry

        lax.fori_loop(0, n_chunks, chunk_body, 0)

    def run(fused, flat_ids):
        ids_r = flat_ids.reshape(NW, n_chunks, CHUNK)
        return gather_k(fused, ids_r)

    return run


def kernel(ids, table, A, Bm):
    V, H = table.shape
    B, S = ids.shape
    fused = _fuse_table(table, A, Bm)
    gather = _make_gather(V, H, B * S)
    out = gather(fused, ids.reshape(-1))
    return out.reshape(B, S, H)


# speedup vs baseline: 5.0825x; 1.0888x over previous
import functools

import jax
import jax.numpy as jnp
from jax import lax
from jax.experimental import pallas as pl
from jax.experimental.pallas import tpu as pltpu
from jax.experimental.pallas import tpu_sc as plsc

SCALE = 0.5
CHUNK = 128
NBUF = 2


def _fuse_table(table, A, Bm):
    V, H = table.shape
    R = A.shape[1]
    blk = 2000
    assert V % blk == 0

    def body(t_ref, a_ref, b_ref, o_ref):
        delta = jax.lax.dot_general(
            a_ref[...], b_ref[...], (((1,), (0,)), ((), ())),
            preferred_element_type=jnp.float32)
        o_ref[...] = t_ref[...] + delta * SCALE

    return pl.pallas_call(
        body,
        grid=(V // blk,),
        in_specs=[
            pl.BlockSpec((blk, H), lambda i: (i, 0)),
            pl.BlockSpec((blk, R), lambda i: (i, 0)),
            pl.BlockSpec((R, H), lambda i: (0, 0)),
        ],
        out_specs=pl.BlockSpec((blk, H), lambda i: (i, 0)),
        out_shape=jax.ShapeDtypeStruct((V, H), jnp.float32),
    )(table, A, Bm)


@functools.lru_cache(maxsize=None)
def _make_gather(V, H, BT):
    info = plsc.get_sparse_core_info()
    NC, NS = info.num_cores, info.num_subcores
    NW = NC * NS
    assert BT % (NW * CHUNK) == 0
    per_tile = BT // NW
    n_chunks = per_tile // CHUNK
    assert n_chunks % NBUF == 0
    mesh = plsc.VectorSubcoreMesh(core_axis_name="c", subcore_axis_name="s")

    @functools.partial(
        pl.kernel,
        mesh=mesh,
        out_type=jax.ShapeDtypeStruct((BT, H), jnp.float32),
        scratch_types=[
            pltpu.VMEM((per_tile,), jnp.int32),
            pltpu.VMEM((NBUF, CHUNK, H), jnp.float32),
        ]
        + [pltpu.SemaphoreType.DMA] * (2 * NBUF),
    )
    def gather_k(fused_hbm, ids_hbm, out_hbm, idx_v, rows_v, *sems):
        gsem = sems[:NBUF]
        ssem = sems[NBUF:]
        c = lax.axis_index("c")
        s = lax.axis_index("s")
        wid = s * NC + c
        base = wid * per_tile
        pltpu.sync_copy(ids_hbm.at[pl.ds(base, per_tile)], idx_v)

        def start_gather(j, b):
            pltpu.async_copy(
                fused_hbm.at[idx_v.at[pl.ds(j * CHUNK, CHUNK)]],
                rows_v.at[b], gsem[b])

        def start_store(j, b):
            pltpu.async_copy(
                rows_v.at[b],
                out_hbm.at[pl.ds(base + j * CHUNK, CHUNK)], ssem[b])

        def wait_gather(b):
            pltpu.make_async_copy(
                fused_hbm.at[pl.ds(0, CHUNK)], rows_v.at[b], gsem[b]).wait()

        def wait_store(b):
            pltpu.make_async_copy(
                rows_v.at[b], out_hbm.at[pl.ds(base, CHUNK)], ssem[b]).wait()

        for b in range(NBUF):
            start_gather(b, b)

        def outer(j0, carry):
            for b in range(NBUF):
                j = j0 + b
                wait_gather(b)
                start_store(j, b)

                @pl.when(j + NBUF < n_chunks)
                def _():
                    wait_store(b)
                    start_gather(j + NBUF, b)

            return carry

        lax.fori_loop(0, n_chunks // NBUF, lambda i, c: outer(i * NBUF, c), 0)

        for b in range(NBUF):
            wait_store(b)

    def run(fused, flat_ids):
        return gather_k(fused, flat_ids)

    return run


def kernel(ids, table, A, Bm):
    V, H = table.shape
    B, S = ids.shape
    fused = _fuse_table(table, A, Bm)
    gather = _make_gather(V, H, B * S)
    out = gather(fused, ids.reshape(-1))
    return out.reshape(B, S, H)


# speedup vs baseline: 6.9602x; 1.3694x over previous
import functools

import jax
import jax.numpy as jnp
from jax import lax
from jax.experimental import pallas as pl
from jax.experimental.pallas import tpu as pltpu
from jax.experimental.pallas import tpu_sc as plsc

SCALE = 0.5
CHUNK = 128
NBUF = 2


def _fuse_table(table, A, Bm):
    V, H = table.shape
    R = A.shape[1]
    blk = 2000
    assert V % blk == 0

    def body(t_ref, a_ref, b_ref, o_ref):
        delta = jax.lax.dot_general(
            a_ref[...], b_ref[...], (((1,), (0,)), ((), ())),
            preferred_element_type=jnp.float32)
        o_ref[...] = t_ref[...] + delta * SCALE

    return pl.pallas_call(
        body,
        grid=(V // blk,),
        in_specs=[
            pl.BlockSpec((blk, H), lambda i: (i, 0)),
            pl.BlockSpec((blk, R), lambda i: (i, 0)),
            pl.BlockSpec((R, H), lambda i: (0, 0)),
        ],
        out_specs=pl.BlockSpec((blk, H), lambda i: (i, 0)),
        out_shape=jax.ShapeDtypeStruct((V, H), jnp.float32),
    )(table, A, Bm)


@functools.lru_cache(maxsize=None)
def _make_gather(V, H, B, S):
    info = plsc.get_sparse_core_info()
    NC, NS = info.num_cores, info.num_subcores
    NW = NC * NS
    assert B % NW == 0
    per_tile = B // NW
    assert per_tile % NBUF == 0
    mesh = plsc.VectorSubcoreMesh(core_axis_name="c", subcore_axis_name="s")

    @functools.partial(
        pl.kernel,
        mesh=mesh,
        out_type=jax.ShapeDtypeStruct((B, S, H), jnp.float32),
        scratch_types=[
            pltpu.VMEM((per_tile, S), jnp.int32),
            pltpu.VMEM((NBUF, S, H), jnp.float32),
        ]
        + [pltpu.SemaphoreType.DMA] * (2 * NBUF),
    )
    def gather_k(fused_hbm, ids_hbm, out_hbm, idx_v, rows_v, *sems):
        gsem = sems[:NBUF]
        ssem = sems[NBUF:]
        c = lax.axis_index("c")
        s = lax.axis_index("s")
        wid = s * NC + c
        base = wid * per_tile
        pltpu.sync_copy(ids_hbm.at[pl.ds(base, per_tile)], idx_v)

        def gather_copy(i, b):
            return pltpu.make_async_copy(
                fused_hbm.at[idx_v.at[i]], rows_v.at[b], gsem[b])

        def store_copy(i, b):
            return pltpu.make_async_copy(
                rows_v.at[b], out_hbm.at[base + i], ssem[b])

        for b in range(NBUF):
            gather_copy(b, b).start()

        def outer(i0, carry):
            for b in range(NBUF):
                i = i0 + b
                gather_copy(i, b).wait()
                store_copy(i, b).start()

                @pl.when(i + NBUF < per_tile)
                def _():
                    store_copy(i, b).wait()
                    gather_copy(i + NBUF, b).start()

            return carry

        lax.fori_loop(0, per_tile // NBUF, lambda i, c: outer(i * NBUF, c), 0)

        for b in range(NBUF):
            store_copy(per_tile - NBUF + b, b).wait()

    return gather_k


def kernel(ids, table, A, Bm):
    V, H = table.shape
    B, S = ids.shape
    fused = _fuse_table(table, A, Bm)
    gather = _make_gather(V, H, B, S)
    return gather(fused, ids)


# speedup vs baseline: 12.6579x; 1.8186x over previous
import functools

import jax
import jax.numpy as jnp
from jax import lax
from jax.experimental import pallas as pl
from jax.experimental.pallas import tpu as pltpu
from jax.experimental.pallas import tpu_sc as plsc

SCALE = 0.5
CHUNK = 128
NBUF = 2


def _fuse_table(table, At, Bm):
    V, H = table.shape
    R = At.shape[0]
    blk = 2048

    def body(t_ref, at_ref, b_ref, o_ref):
        delta = jax.lax.dot_general(
            at_ref[...], b_ref[...], (((0,), (0,)), ((), ())),
            preferred_element_type=jnp.float32)
        o_ref[...] = t_ref[...] + delta * SCALE

    return pl.pallas_call(
        body,
        grid=(pl.cdiv(V, blk),),
        in_specs=[
            pl.BlockSpec((blk, H), lambda i: (i, 0)),
            pl.BlockSpec((R, blk), lambda i: (0, i)),
            pl.BlockSpec((R, H), lambda i: (0, 0)),
        ],
        out_specs=pl.BlockSpec((blk, H), lambda i: (i, 0)),
        out_shape=jax.ShapeDtypeStruct((V, H), jnp.float32),
    )(table, At, Bm)


@functools.lru_cache(maxsize=None)
def _make_gather(V, H, BT):
    info = plsc.get_sparse_core_info()
    NC, NS = info.num_cores, info.num_subcores
    NW = NC * NS
    assert BT % (NW * CHUNK) == 0
    per_tile = BT // NW
    n_chunks = per_tile // CHUNK
    assert n_chunks % NBUF == 0
    mesh = plsc.VectorSubcoreMesh(core_axis_name="c", subcore_axis_name="s")

    @functools.partial(
        pl.kernel,
        mesh=mesh,
        out_type=jax.ShapeDtypeStruct((BT, H), jnp.float32),
        scratch_types=[
            pltpu.VMEM((per_tile,), jnp.int32),
            pltpu.VMEM((NBUF, CHUNK, H), jnp.float32),
        ]
        + [pltpu.SemaphoreType.DMA] * (2 * NBUF),
    )
    def gather_k(fused_hbm, ids_hbm, out_hbm, idx_v, rows_v, *sems):
        gsem = sems[:NBUF]
        ssem = sems[NBUF:]
        c = lax.axis_index("c")
        s = lax.axis_index("s")
        wid = s * NC + c
        base = wid * per_tile
        pltpu.sync_copy(ids_hbm.at[pl.ds(base, per_tile)], idx_v)

        def gather_copy(j, b):
            return pltpu.make_async_copy(
                fused_hbm.at[idx_v.at[pl.ds(j * CHUNK, CHUNK)]],
                rows_v.at[b], gsem[b])

        def store_copy(j, b):
            return pltpu.make_async_copy(
                rows_v.at[b],
                out_hbm.at[pl.ds(base + j * CHUNK, CHUNK)], ssem[b])

        for b in range(NBUF):
            gather_copy(b, b).start()

        def outer(j0, carry):
            for b in range(NBUF):
                j = j0 + b
                gather_copy(j, b).wait()
                store_copy(j, b).start()

                @pl.when(j + NBUF < n_chunks)
                def _():
                    store_copy(j, b).wait()
                    gather_copy(j + NBUF, b).start()

            return carry

        lax.fori_loop(0, n_chunks // NBUF, lambda i, c: outer(i * NBUF, c), 0)

        for b in range(NBUF):
            store_copy(n_chunks - NBUF + b, b).wait()

    return gather_k


def kernel(ids, table, A, Bm):
    V, H = table.shape
    B, S = ids.shape
    fused = _fuse_table(table, A.T, Bm)
    gather = _make_gather(V, H, B * S)
    out_t = gather(fused, ids.T.reshape(-1))
    return out_t.reshape(S, B, H).transpose(1, 0, 2)


# speedup vs baseline: 15.0844x; 1.1917x over previous
import functools

import jax
import jax.numpy as jnp
from jax import lax
from jax.experimental import pallas as pl
from jax.experimental.pallas import tpu as pltpu
from jax.experimental.pallas import tpu_sc as plsc

SCALE = 0.5
CHUNK = 128
NBUF = 5


def _fuse_table(table, At, Bm):
    V, H = table.shape
    R = At.shape[0]
    blk = 8192

    def body(t_ref, at_ref, b_ref, o_ref):
        delta = jax.lax.dot_general(
            at_ref[...], b_ref[...], (((0,), (0,)), ((), ())),
            preferred_element_type=jnp.float32)
        o_ref[...] = t_ref[...] + delta * SCALE

    return pl.pallas_call(
        body,
        grid=(pl.cdiv(V, blk),),
        in_specs=[
            pl.BlockSpec((blk, H), lambda i: (i, 0)),
            pl.BlockSpec((R, blk), lambda i: (0, i)),
            pl.BlockSpec((R, H), lambda i: (0, 0)),
        ],
        out_specs=pl.BlockSpec((blk, H), lambda i: (i, 0)),
        out_shape=jax.ShapeDtypeStruct((V, H), jnp.float32),
    )(table, At, Bm)


@functools.lru_cache(maxsize=None)
def _make_gather(V, H, BT):
    info = plsc.get_sparse_core_info()
    NC, NS = info.num_cores, info.num_subcores
    NW = NC * NS
    assert BT % (NW * CHUNK) == 0
    per_tile = BT // NW
    n_chunks = per_tile // CHUNK
    assert n_chunks % NBUF == 0
    mesh = plsc.VectorSubcoreMesh(core_axis_name="c", subcore_axis_name="s")

    @functools.partial(
        pl.kernel,
        mesh=mesh,
        out_type=jax.ShapeDtypeStruct((BT, H), jnp.float32),
        scratch_types=[
            pltpu.VMEM((per_tile,), jnp.int32),
            pltpu.VMEM((NBUF, CHUNK, H), jnp.float32),
        ]
        + [pltpu.SemaphoreType.DMA] * (2 * NBUF),
    )
    def gather_k(fused_hbm, ids_hbm, out_hbm, idx_v, rows_v, *sems):
        gsem = sems[:NBUF]
        ssem = sems[NBUF:]
        c = lax.axis_index("c")
        s = lax.axis_index("s")
        wid = s * NC + c
        base = wid * per_tile
        pltpu.sync_copy(ids_hbm.at[pl.ds(base, per_tile)], idx_v)

        def gather_copy(j, b):
            return pltpu.make_async_copy(
                fused_hbm.at[idx_v.at[pl.ds(j * CHUNK, CHUNK)]],
                rows_v.at[b], gsem[b])

        def store_copy(j, b):
            return pltpu.make_async_copy(
                rows_v.at[b],
                out_hbm.at[pl.ds(base + j * CHUNK, CHUNK)], ssem[b])

        for b in range(NBUF):
            gather_copy(b, b).start()

        def outer(j0, carry):
            for b in range(NBUF):
                j = j0 + b
                gather_copy(j, b).wait()
                store_copy(j, b).start()

                @pl.when(j + NBUF < n_chunks)
                def _():
                    store_copy(j, b).wait()
                    gather_copy(j + NBUF, b).start()

            return carry

        lax.fori_loop(0, n_chunks // NBUF, lambda i, c: outer(i * NBUF, c), 0)

        for b in range(NBUF):
            store_copy(n_chunks - NBUF + b, b).wait()

    return gather_k


def kernel(ids, table, A, Bm):
    V, H = table.shape
    B, S = ids.shape
    fused = _fuse_table(table, A.T, Bm)
    gather = _make_gather(V, H, B * S)
    out_t = gather(fused, ids.T.reshape(-1))
    return out_t.reshape(S, B, H).transpose(1, 0, 2)
